# baseline (speedup 1.0000x reference)
import jax
import jax.numpy as jnp
from jax import lax
from jax.experimental import pallas as pl
from jax.experimental.pallas import tpu as pltpu
from jax.experimental.pallas import tpu_sc as plsc

B = 16384
F = 26
V = 100000
E = 64

NC = 2
NS = 16
NW = NC * NS

ROWS = B * F
IDX_ROWS = ROWS // 128
IDX_PER_W = IDX_ROWS // NW

WL = 512
WPF = V // WL
TAIL0 = WPF * WL
NWIN = F * WPF
WIN_PER_W = -(-NWIN // NW)


def _transpose_pairs(staged, out_v, nlanes):
    iot = lax.iota(jnp.int32, 16)

    def pair_block(p0, _):
        for dp in range(8):
            p = p0 * 8 + dp
            for g in range(8):
                h = g // 4
                rows = 16 * (g % 4) + iot
                cols = jnp.full((16,), 2 * p + h, dtype=jnp.int32)
                out_v[p, pl.ds(16 * g, 16)] = \
                    plsc.load_gather(staged, [rows, cols])
        return 0

    lax.fori_loop(0, nlanes // 16, pair_block, 0)


def _sc_repack_body(tt_hbm, tail_hbm, out_hbm, stg_a, stg_b, out_v,
                    sem0, sem1):
    wid = lax.axis_index("s") * NC + lax.axis_index("c")

    def win_addr(gw):
        f = lax.div(gw, WPF)
        wv = lax.rem(gw, WPF)
        return f, wv * WL

    def start(gw, stg, sem):
        f, v0 = win_addr(gw)
        pltpu.async_copy(tt_hbm.at[f, :, pl.ds(v0, WL)],
                         stg.at[:, pl.ds(0, WL)], sem)

    def wait(gw, stg, sem):
        f, v0 = win_addr(gw)
        pltpu.make_async_copy(tt_hbm.at[f, :, pl.ds(v0, WL)],
                              stg.at[:, pl.ds(0, WL)], sem).wait()

    def process(gw, stg):
        f, v0 = win_addr(gw)
        _transpose_pairs(stg, out_v, WL)
        p0 = pl.multiple_of(f * (V // 2) + v0 // 2, 16)
        pltpu.sync_copy(out_v, out_hbm.at[pl.ds(p0, WL // 2)])

    start(wid, stg_a, sem0)

    def step(k, _):
        gw = wid + NW * k
        buf = lax.rem(k, 2)
        nxt = gw + NW

        @pl.when(nxt < NWIN)
        def _():
            lax.cond(buf == 0,
                     lambda: start(nxt, stg_b, sem1),
                     lambda: start(nxt, stg_a, sem0))

        @pl.when(gw < NWIN)
        def _():
            def b0():
                wait(gw, stg_a, sem0)
                process(gw, stg_a)

            def b1():
                wait(gw, stg_b, sem1)
                process(gw, stg_b)

            lax.cond(buf == 0, b0, b1)
        return 0

    lax.fori_loop(0, WIN_PER_W, step, 0)

    @pl.when(wid < F)
    def _():
        f = wid
        pltpu.sync_copy(tt_hbm.at[f, :, pl.ds(TAIL0, 128)],
                        stg_a.at[:, pl.ds(0, 128)])
        _transpose_pairs(stg_a, out_v, 128)
        pltpu.sync_copy(
            out_v.at[pl.ds(0, 64)],
            out_hbm.at[pl.ds(pl.multiple_of(f * (V // 2) + TAIL0 // 2, 16),
                             64)])
        pltpu.sync_copy(tail_hbm.at[pl.ds(pl.multiple_of(16 * f, 16), 16)],
                        out_v.at[pl.ds(0, 16)])
        pltpu.sync_copy(
            out_v.at[pl.ds(0, 16)],
            out_hbm.at[pl.ds(
                pl.multiple_of(f * (V // 2) + TAIL0 // 2 + 64, 16), 16)])


def _sc_repack(t_T, tail128):
    mesh = plsc.VectorSubcoreMesh(core_axis_name="c", subcore_axis_name="s",
                                  num_cores=NC, num_subcores=NS)
    return pl.kernel(
        _sc_repack_body,
        out_type=jax.ShapeDtypeStruct((F * V // 2, 128), jnp.float32),
        mesh=mesh,
        compiler_params=pltpu.CompilerParams(use_tc_tiling_on_sc=True,
                                             needs_layout_passes=False),
        scratch_types=[
            pltpu.VMEM((E, WL + 1), jnp.float32),
            pltpu.VMEM((E, WL + 1), jnp.float32),
            pltpu.VMEM((WL // 2, 128), jnp.float32),
            pltpu.SemaphoreType.DMA,
            pltpu.SemaphoreType.DMA,
        ],
    )(t_T, tail128)


def _sc_gather_body(table_hbm, idx_hbm, out_hbm, idx_v, rows_v, sem0, sem1):
    wid = lax.axis_index("s") * NC + lax.axis_index("c")
    row_base = wid * IDX_PER_W
    pltpu.sync_copy(idx_hbm.at[pl.ds(row_base, IDX_PER_W)], idx_v)

    def start(j, buf, sem):
        pltpu.async_copy(table_hbm.at[idx_v.at[j]], rows_v.at[buf], sem)

    def drain_write(j, buf, sem):
        pltpu.make_async_copy(table_hbm.at[idx_v.at[j]], rows_v.at[buf],
                              sem).wait()
        pltpu.sync_copy(rows_v.at[buf],
                        out_hbm.at[pl.ds((row_base + j) * 128, 128)])

    start(0, 0, sem0)

    def step(j, _):
        buf = lax.rem(j, 2)

        @pl.when(j + 1 < IDX_PER_W)
        def _():
            lax.cond(buf == 0,
                     lambda: start(j + 1, 1, sem1),
                     lambda: start(j + 1, 0, sem0))
        lax.cond(buf == 0,
                 lambda: drain_write(j, 0, sem0),
                 lambda: drain_write(j, 1, sem1))
        return 0

    lax.fori_loop(0, IDX_PER_W, step, 0)


def _sc_gather(t128, idxp):
    mesh = plsc.VectorSubcoreMesh(core_axis_name="c", subcore_axis_name="s",
                                  num_cores=NC, num_subcores=NS)
    return pl.kernel(
        _sc_gather_body,
        out_type=jax.ShapeDtypeStruct((ROWS, 128), jnp.float32),
        mesh=mesh,
        compiler_params=pltpu.CompilerParams(use_tc_tiling_on_sc=True),
        scratch_types=[
            pltpu.VMEM((IDX_PER_W, 128), jnp.int32),
            pltpu.VMEM((2, 128, 128), jnp.float32),
            pltpu.SemaphoreType.DMA,
            pltpu.SemaphoreType.DMA,
        ],
    )(t128, idxp)


BT = 512


def _tc_body(praw_ref, xc_ref, xn_ref, w2_ref, wn_ref, bn_ref, wf_ref, bf_ref,
             out_ref):
    num = jnp.dot(xn_ref[...], wn_ref[...],
                  preferred_element_type=jnp.float32) + bn_ref[...]
    acc = jnp.dot(num, wf_ref[...], preferred_element_type=jnp.float32)
    acc += bf_ref[...]

    praw = praw_ref[...].reshape(BT, F, 128)
    half = (xc_ref[...] & 1).astype(jnp.int32)
    lane = lax.broadcasted_iota(jnp.int32, (BT, F, 128), 2) // 64
    masked = jnp.where(lane == half[:, :, None], praw, 0.0)
    for f in range(F):
        acc += jnp.dot(masked[:, f, :], w2_ref[f],
                       preferred_element_type=jnp.float32)
    out_ref[...] = acc


def _tc_matmul(praw, X_cat, X_num, W2, W_num, b_num, W_fnum, b_final):
    grid = (B // BT,)
    nnf = X_num.shape[1]
    return pl.pallas_call(
        _tc_body,
        grid=grid,
        in_specs=[
            pl.BlockSpec((BT * F, 128), lambda i: (i, 0)),
            pl.BlockSpec((BT, F), lambda i: (i, 0)),
            pl.BlockSpec((BT, nnf), lambda i: (i, 0)),
            pl.BlockSpec((F, 128, E), lambda i: (0, 0, 0)),
            pl.BlockSpec((nnf, E), lambda i: (0, 0)),
            pl.BlockSpec((1, E), lambda i: (0, 0)),
            pl.BlockSpec((E, E), lambda i: (0, 0)),
            pl.BlockSpec((1, E), lambda i: (0, 0)),
        ],
        out_specs=pl.BlockSpec((BT, E), lambda i: (i, 0)),
        out_shape=jax.ShapeDtypeStruct((B, E), jnp.float32),
    )(praw, X_cat, X_num, W2, W_num, b_num, W_fnum, b_final)


def kernel(X_cat, X_num, tables, W_num, b_num, W_final, b_final):
    t_T = jnp.transpose(tables, (0, 2, 1))
    tail128 = tables[:, TAIL0 + 128:, :].reshape(F * 16, 128)
    t128 = _sc_repack(t_T, tail128)
    xc = X_cat.astype(jnp.int32)
    flat = xc + (jnp.arange(F, dtype=jnp.int32) * V)[None, :]
    idxp = (flat >> 1).reshape(IDX_ROWS, 128)
    praw = _sc_gather(t128, idxp)

    W_cat = W_final[:F * E].reshape(F, E, E)
    W2 = jnp.concatenate([W_cat, W_cat], axis=1)
    W_fnum = W_final[F * E:]
    return _tc_matmul(praw, xc, X_num, W2, W_num,
                      b_num.reshape(1, E), W_fnum, b_final.reshape(1, E))


# speedup vs baseline: 2.5628x; 2.5628x over previous
import jax
import jax.numpy as jnp
from jax import lax
from jax.experimental import pallas as pl
from jax.experimental.pallas import tpu as pltpu
from jax.experimental.pallas import tpu_sc as plsc

B = 16384
F = 26
V = 100000
E = 64

NC = 2
NS = 16
NW = NC * NS

ROWS = B * F
IDX_ROWS = ROWS // 128
IDX_PER_W = IDX_ROWS // NW


def _sc_gather_body(table_hbm, idx_hbm, out_hbm, idx_v, rows_v, sem0, sem1):
    wid = lax.axis_index("s") * NC + lax.axis_index("c")
    row_base = wid * IDX_PER_W
    pltpu.sync_copy(idx_hbm.at[pl.ds(row_base, IDX_PER_W)], idx_v)

    def start(j, buf, sem):
        pltpu.async_copy(table_hbm.at[idx_v.at[j]], rows_v.at[buf], sem)

    def drain_write(j, buf, sem):
        pltpu.make_async_copy(table_hbm.at[idx_v.at[j]], rows_v.at[buf], sem).wait()
        pltpu.sync_copy(rows_v.at[buf], out_hbm.at[pl.ds((row_base + j) * 128, 128)])

    start(0, 0, sem0)

    def step(j, _):
        buf = lax.rem(j, 2)

        @pl.when(j + 1 < IDX_PER_W)
        def _():
            lax.cond(buf == 0,
                     lambda: start(j + 1, 1, sem1),
                     lambda: start(j + 1, 0, sem0))
        lax.cond(buf == 0,
                 lambda: drain_write(j, 0, sem0),
                 lambda: drain_write(j, 1, sem1))
        return 0

    lax.fori_loop(0, IDX_PER_W, step, 0)


def _sc_gather(tables2, idx2d):
    mesh = plsc.VectorSubcoreMesh(core_axis_name="c", subcore_axis_name="s",
                                  num_cores=NC, num_subcores=NS)
    return pl.kernel(
        _sc_gather_body,
        out_type=jax.ShapeDtypeStruct((ROWS, E), jnp.float32),
        mesh=mesh,
        compiler_params=pltpu.CompilerParams(use_tc_tiling_on_sc=False),
        scratch_types=[
            pltpu.VMEM((IDX_PER_W, 128), jnp.int32),
            pltpu.VMEM((2, 128, E), jnp.float32),
            pltpu.SemaphoreType.DMA,
            pltpu.SemaphoreType.DMA,
        ],
    )(tables2, idx2d)


BT = 512


def _tc_body(g_ref, xn_ref, wc_ref, wn_ref, bn_ref, wf_ref, bf_ref, out_ref):
    num = jnp.dot(xn_ref[...], wn_ref[...],
                  preferred_element_type=jnp.float32) + bn_ref[...]
    acc = jnp.dot(g_ref[...], wc_ref[...], preferred_element_type=jnp.float32)
    acc += jnp.dot(num, wf_ref[...], preferred_element_type=jnp.float32)
    out_ref[...] = acc + bf_ref[...]


def _tc_matmul(g, X_num, W_cat, W_num, b_num, W_fnum, b_final):
    grid = (B // BT,)
    return pl.pallas_call(
        _tc_body,
        grid=grid,
        in_specs=[
            pl.BlockSpec((BT, F * E), lambda i: (i, 0)),
            pl.BlockSpec((BT, X_num.shape[1]), lambda i: (i, 0)),
            pl.BlockSpec((F * E, E), lambda i: (0, 0)),
            pl.BlockSpec((X_num.shape[1], E), lambda i: (0, 0)),
            pl.BlockSpec((1, E), lambda i: (0, 0)),
            pl.BlockSpec((E, E), lambda i: (0, 0)),
            pl.BlockSpec((1, E), lambda i: (0, 0)),
        ],
        out_specs=pl.BlockSpec((BT, E), lambda i: (i, 0)),
        out_shape=jax.ShapeDtypeStruct((B, E), jnp.float32),
    )(g, X_num, W_cat, W_num, b_num, W_fnum, b_final)


def kernel(X_cat, X_num, tables, W_num, b_num, W_final, b_final):
    tables2 = tables.reshape(F * V, E)
    idx_flat = (X_cat.astype(jnp.int32)
                + (jnp.arange(F, dtype=jnp.int32) * V)[None, :])
    idx2d = idx_flat.reshape(IDX_ROWS, 128)
    g = _sc_gather(tables2, idx2d).reshape(B, F * E)
    W_cat = W_final[:F * E]
    W_fnum = W_final[F * E:]
    out = _tc_matmul(g, X_num, W_cat, W_num,
                     b_num.reshape(1, E), W_fnum, b_final.reshape(1, E))
    return out
